# baseline (speedup 1.0000x reference)
import jax
import jax.numpy as jnp
from jax import lax
from jax.experimental import pallas as pl
from jax.experimental.pallas import tpu as pltpu
from jax.experimental.pallas import tpu_sc as plsc

_L = 16
_CHUNK = 64


def _lane_sum16(v):
    lanes = lax.iota(jnp.int32, _L)
    dnums = lax.GatherDimensionNumbers(
        offset_dims=(), collapsed_slice_dims=(0,), start_index_map=(0,))
    for k in (8, 4, 2, 1):
        shuf = lax.gather(
            v, (lanes ^ k)[:, None], dimension_numbers=dnums, slice_sizes=(1,),
            mode=lax.GatherScatterMode.PROMISE_IN_BOUNDS)
        v = v + shuf
    return v


def _rsqrt16(v):
    s = (v + 1.0) * 0.5
    for _ in range(26):
        s = (s + v / s) * 0.5
    return 1.0 / s


def kernel(x, pos_table, gamma, beta):
    B, S, D = x.shape
    N = B * S
    n_lane_slices = D // _L

    info = plsc.get_sparse_core_info()
    nw = info.num_cores * info.num_subcores
    rows_per_w = N // nw
    n_chunks = rows_per_w // _CHUNK

    x_flat = x.reshape(N * D)
    pos_flat = pos_table.reshape(S * D)

    mesh = plsc.VectorSubcoreMesh(core_axis_name="c", subcore_axis_name="s")

    @pl.kernel(
        mesh=mesh,
        out_type=jax.ShapeDtypeStruct((N * D,), jnp.float32),
        scratch_types=[
            pltpu.VMEM((_CHUNK * D,), jnp.float32),
            pltpu.VMEM((_CHUNK * D,), jnp.float32),
            pltpu.VMEM((D,), jnp.float32),
            pltpu.VMEM((D,), jnp.float32),
        ],
    )
    def sc_k(x_hbm, pos_hbm, gamma_hbm, beta_hbm, out_hbm, xbuf, pbuf, gbuf, bbuf):
        wid = lax.axis_index("s") * info.num_cores + lax.axis_index("c")
        pltpu.sync_copy(gamma_hbm, gbuf)
        pltpu.sync_copy(beta_hbm, bbuf)

        def chunk_body(c, _):
            row0 = wid * rows_per_w + c * _CHUNK
            prow0 = lax.rem(row0, S)
            pltpu.sync_copy(x_hbm.at[pl.ds(row0 * D, _CHUNK * D)], xbuf)
            pltpu.sync_copy(pos_hbm.at[pl.ds(prow0 * D, _CHUNK * D)], pbuf)

            def row_body(r, _):
                base = r * D
                acc = jnp.zeros((_L,), jnp.float32)
                accsq = jnp.zeros((_L,), jnp.float32)
                for i in range(n_lane_slices):
                    sl = pl.ds(base + i * _L, _L)
                    e = xbuf[sl] + pbuf[sl]
                    xbuf[sl] = e
                    acc = acc + e
                    accsq = accsq + e * e
                mean16 = _lane_sum16(acc) * (1.0 / D)
                meansq16 = _lane_sum16(accsq) * (1.0 / D)
                var16 = meansq16 - mean16 * mean16
                inv16 = _rsqrt16(var16 + 1e-12)
                for i in range(n_lane_slices):
                    sl = pl.ds(base + i * _L, _L)
                    gsl = pl.ds(i * _L, _L)
                    xbuf[sl] = (xbuf[sl] - mean16) * (inv16 * gbuf[gsl]) + bbuf[gsl]
                return 0

            lax.fori_loop(0, _CHUNK, row_body, 0)
            pltpu.sync_copy(xbuf, out_hbm.at[pl.ds(row0 * D, _CHUNK * D)])
            return 0

        lax.fori_loop(0, n_chunks, chunk_body, 0)

    out = sc_k(x_flat, pos_flat, gamma, beta)
    return out.reshape(B, S, D)


# speedup vs baseline: 13.4239x; 13.4239x over previous
import jax
import jax.numpy as jnp
from jax.experimental import pallas as pl
from jax.experimental.pallas import tpu as pltpu

_SEQ_TILE = 2048


def _ln_kernel(x_ref, pos_ref, gamma_ref, beta_ref, out_ref):
    s = pl.program_id(0)
    ts = x_ref.shape[1]
    e = x_ref[0] + pos_ref[pl.ds(s * ts, ts), :]
    d_inv = 1.0 / e.shape[1]
    mean = jnp.sum(e, axis=1, keepdims=True) * d_inv
    meansq = jnp.sum(e * e, axis=1, keepdims=True) * d_inv
    var = meansq - mean * mean
    inv = jax.lax.rsqrt(var + 1e-12)
    out_ref[0] = (e - mean) * (inv * gamma_ref[...]) + beta_ref[...]


def kernel(x, pos_table, gamma, beta):
    B, S, D = x.shape
    ts = _SEQ_TILE
    gamma2 = gamma.reshape(1, D)
    beta2 = beta.reshape(1, D)
    grid = (S // ts, B)
    return pl.pallas_call(
        _ln_kernel,
        grid=grid,
        in_specs=[
            pl.BlockSpec((1, ts, D), lambda s, b: (b, s, 0)),
            pl.BlockSpec((S, D), lambda s, b: (0, 0)),
            pl.BlockSpec((1, D), lambda s, b: (0, 0)),
            pl.BlockSpec((1, D), lambda s, b: (0, 0)),
        ],
        out_specs=pl.BlockSpec((1, ts, D), lambda s, b: (b, s, 0)),
        out_shape=jax.ShapeDtypeStruct((B, S, D), x.dtype),
        compiler_params=pltpu.CompilerParams(
            dimension_semantics=("parallel", "parallel"),
        ),
    )(x, pos_table, gamma2, beta2)


# speedup vs baseline: 13.7342x; 1.0231x over previous
import jax
import jax.numpy as jnp
from jax.experimental import pallas as pl
from jax.experimental.pallas import tpu as pltpu

_SEQ_TILE = 2048


def _ln_kernel(x_ref, pos_ref, gamma_ref, beta_ref, out_ref):
    s = pl.program_id(0)
    ts = x_ref.shape[1]
    e = x_ref[0] + pos_ref[pl.ds(s * ts, ts), :]
    d_inv = 1.0 / e.shape[1]
    mean = jnp.sum(e, axis=1, keepdims=True) * d_inv
    meansq = jnp.sum(e * e, axis=1, keepdims=True) * d_inv
    var = meansq - mean * mean
    inv = jax.lax.rsqrt(var + 1e-12)
    out_ref[0] = (e - mean) * (inv * gamma_ref[...]) + beta_ref[...]


def kernel(x, pos_table, gamma, beta):
    B, S, D = x.shape
    ts = _SEQ_TILE
    gamma2 = gamma.reshape(1, D)
    beta2 = beta.reshape(1, D)
    grid = (S // ts, B)
    return pl.pallas_call(
        _ln_kernel,
        grid=grid,
        in_specs=[
            pl.BlockSpec((1, ts, D), lambda s, b: (b, s, 0)),
            pl.BlockSpec((S, D), lambda s, b: (0, 0)),
            pl.BlockSpec((1, D), lambda s, b: (0, 0)),
            pl.BlockSpec((1, D), lambda s, b: (0, 0)),
        ],
        out_specs=pl.BlockSpec((1, ts, D), lambda s, b: (b, s, 0)),
        out_shape=jax.ShapeDtypeStruct((B, S, D), x.dtype),
        compiler_params=pltpu.CompilerParams(
            dimension_semantics=("arbitrary", "arbitrary"),
        ),
    )(x, pos_table, gamma2, beta2)
